# Initial kernel scaffold; baseline (speedup 1.0000x reference)
import functools

import jax
import jax.numpy as jnp
from jax import lax
from jax.experimental import pallas as pl
from jax.experimental.pallas import tpu as pltpu
from jax.experimental.pallas import tpu_sc as plsc

N = 10000
D = 128
NC = 2
NS = 16
NW = NC * NS
B = 128
N_PAD = 10240
ROWS = N_PAD // NS


def _make_aggregate(chunks, with_counts):
    out_type = [jax.ShapeDtypeStruct((NC, N_PAD, D), jnp.float32)]
    scratch = [
        pltpu.VMEM_SHARED((N_PAD, D), jnp.float32),
        pltpu.VMEM((chunks, B), jnp.int32),
        pltpu.VMEM((chunks, B), jnp.int32),
        pltpu.VMEM((B, D), jnp.float32),
        pltpu.SemaphoreType.DMA,
    ]
    if with_counts:
        out_type.append(jax.ShapeDtypeStruct((NC, N_PAD, 16), jnp.float32))
        scratch += [
            pltpu.VMEM_SHARED((N_PAD, 16), jnp.float32),
            pltpu.VMEM((B, 16), jnp.float32),
        ]
    mesh = plsc.VectorSubcoreMesh(core_axis_name="c", subcore_axis_name="s")

    if with_counts:
        def body(feats, srci, dsti, zf, zc, ones_in,
                 psum, pcnt, acc, srcv, dstv, rowsv, sem, cacc, onesv):
            c = lax.axis_index("c")
            s = lax.axis_index("s")
            wid = c * NS + s
            pltpu.sync_copy(zf, acc.at[pl.ds(s * ROWS, ROWS)])
            pltpu.sync_copy(zc, cacc.at[pl.ds(s * ROWS, ROWS)])
            pltpu.sync_copy(ones_in, onesv)
            pltpu.sync_copy(srci.at[wid], srcv)
            pltpu.sync_copy(dsti.at[wid], dstv)
            plsc.subcore_barrier()

            def step(j, carry):
                pltpu.async_copy(feats.at[srcv.at[j]], rowsv, sem).wait()
                pltpu.sync_copy(rowsv, acc.at[dstv.at[j]], add=True)
                pltpu.sync_copy(onesv, cacc.at[dstv.at[j]], add=True)
                return carry

            lax.fori_loop(0, chunks, step, 0)
            plsc.subcore_barrier()
            pltpu.sync_copy(acc.at[pl.ds(s * ROWS, ROWS)],
                            psum.at[c].at[pl.ds(s * ROWS, ROWS)])
            pltpu.sync_copy(cacc.at[pl.ds(s * ROWS, ROWS)],
                            pcnt.at[c].at[pl.ds(s * ROWS, ROWS)])
    else:
        def body(feats, srci, dsti, zf,
                 psum, acc, srcv, dstv, rowsv, sem):
            c = lax.axis_index("c")
            s = lax.axis_index("s")
            wid = c * NS + s
            pltpu.sync_copy(zf, acc.at[pl.ds(s * ROWS, ROWS)])
            pltpu.sync_copy(srci.at[wid], srcv)
            pltpu.sync_copy(dsti.at[wid], dstv)
            plsc.subcore_barrier()

            def step(j, carry):
                pltpu.async_copy(feats.at[srcv.at[j]], rowsv, sem).wait()
                pltpu.sync_copy(rowsv, acc.at[dstv.at[j]], add=True)
                return carry

            lax.fori_loop(0, chunks, step, 0)
            plsc.subcore_barrier()
            pltpu.sync_copy(acc.at[pl.ds(s * ROWS, ROWS)],
                            psum.at[c].at[pl.ds(s * ROWS, ROWS)])

    return pl.kernel(body, out_type=out_type, mesh=mesh, scratch_types=scratch)


def _dense_body(act, p0r, p1r, c0r, c1r, xr, wlr, blr, wrr, outr):
    cnt = c0r[...][:, 0:1] + c1r[...][:, 0:1]
    inv = 1.0 / jnp.maximum(cnt, 1.0)
    mean = (p0r[...] + p1r[...]) * inv
    y = (jnp.dot(mean, wlr[...], preferred_element_type=jnp.float32)
         + jnp.dot(xr[...], wrr[...], preferred_element_type=jnp.float32)
         + blr[...])
    if act:
        y = jnp.where(y > 0.0, y, jnp.expm1(y))
    outr[...] = y


def _dense(p0, p1, c0, c1, x, Wl, bl, Wr, act):
    bn = 1000
    grid = (N // bn,)
    row_spec = pl.BlockSpec((bn, D), lambda i: (i, 0))
    cnt_spec = pl.BlockSpec((bn, 16), lambda i: (i, 0))
    w_spec = pl.BlockSpec((D, D), lambda i: (0, 0))
    b_spec = pl.BlockSpec((1, D), lambda i: (0, 0))
    return pl.pallas_call(
        functools.partial(_dense_body, act),
        grid=grid,
        in_specs=[row_spec, row_spec, cnt_spec, cnt_spec, row_spec,
                  w_spec, b_spec, w_spec],
        out_specs=row_spec,
        out_shape=jax.ShapeDtypeStruct((N, D), jnp.float32),
    )(p0, p1, c0, c1, x, Wl, bl.reshape(1, D), Wr)


def kernel(x, edge_index, W1l, b1l, W1r, W2l, b2l, W2r):
    src = edge_index[0]
    dst = edge_index[1]
    e = src.shape[0]
    chunks = -(-e // (NW * B))
    pad = chunks * NW * B - e
    if pad:
        src = jnp.concatenate([src, jnp.zeros((pad,), jnp.int32)])
        dst = jnp.concatenate([dst, jnp.full((pad,), N, jnp.int32)])
    src3 = src.reshape(NW, chunks, B)
    dst3 = dst.reshape(NW, chunks, B)
    zf = jnp.zeros((ROWS, D), jnp.float32)
    zc = jnp.zeros((ROWS, 16), jnp.float32)
    ones = jnp.ones((B, 16), jnp.float32)

    agg_counts = _make_aggregate(chunks, with_counts=True)
    agg_plain = _make_aggregate(chunks, with_counts=False)

    psum, pcnt = agg_counts(x, src3, dst3, zf, zc, ones)
    c0 = pcnt[0, :N]
    c1 = pcnt[1, :N]
    h = _dense(psum[0, :N], psum[1, :N], c0, c1, x, W1l, b1l, W1r, act=True)
    (psum2,) = agg_plain(h, src3, dst3, zf)
    return _dense(psum2[0, :N], psum2[1, :N], c0, c1, h, W2l, b2l, W2r,
                  act=False)


# speedup vs baseline: 4.6218x; 4.6218x over previous
import functools

import jax
import jax.numpy as jnp
from jax import lax
from jax.experimental import pallas as pl
from jax.experimental.pallas import tpu as pltpu
from jax.experimental.pallas import tpu_sc as plsc

N = 10000
D = 128
NC = 2
NS = 16
NW = NC * NS
B = 128
N_PAD = 10240
ROWS = N_PAD // NS


def _make_aggregate(chunks):
    mesh = plsc.VectorSubcoreMesh(core_axis_name="c", subcore_axis_name="s")

    def body(feats, srci, dsti, zf, psum, acc, srcv, dstv, rowsv, sem):
        c = lax.axis_index("c")
        s = lax.axis_index("s")
        wid = c * NS + s
        pltpu.sync_copy(zf, acc.at[pl.ds(s * ROWS, ROWS)])
        pltpu.sync_copy(srci.at[wid], srcv)
        pltpu.sync_copy(dsti.at[wid], dstv)
        plsc.subcore_barrier()

        def step(j, carry):
            pltpu.async_copy(feats.at[srcv.at[j]], rowsv, sem).wait()
            pltpu.sync_copy(rowsv, acc.at[dstv.at[j]], add=True)
            return carry

        lax.fori_loop(0, chunks, step, 0)
        plsc.subcore_barrier()
        pltpu.sync_copy(acc.at[pl.ds(s * ROWS, ROWS)],
                        psum.at[c].at[pl.ds(s * ROWS, ROWS)])

    return pl.kernel(
        body,
        out_type=jax.ShapeDtypeStruct((NC, N_PAD, D), jnp.float32),
        mesh=mesh,
        scratch_types=[
            pltpu.VMEM_SHARED((N_PAD, D), jnp.float32),
            pltpu.VMEM((chunks, B), jnp.int32),
            pltpu.VMEM((chunks, B), jnp.int32),
            pltpu.VMEM((B, D), jnp.float32),
            pltpu.SemaphoreType.DMA,
        ],
    )


def _make_counts(chunks):
    mesh = plsc.VectorSubcoreMesh(core_axis_name="c", subcore_axis_name="s")

    def body(dsti, zc, ones_in, pcnt, cacc, dstv, onesv):
        c = lax.axis_index("c")
        s = lax.axis_index("s")
        wid = c * NS + s
        pltpu.sync_copy(zc, cacc.at[pl.ds(s * ROWS, ROWS)])
        pltpu.sync_copy(ones_in, onesv)
        pltpu.sync_copy(dsti.at[wid], dstv)
        plsc.subcore_barrier()

        def step(j, carry):
            pltpu.sync_copy(onesv, cacc.at[dstv.at[j]], add=True)
            return carry

        lax.fori_loop(0, chunks, step, 0)
        plsc.subcore_barrier()
        pltpu.sync_copy(cacc.at[pl.ds(s * ROWS, ROWS)],
                        pcnt.at[c].at[pl.ds(s * ROWS, ROWS)])

    return pl.kernel(
        body,
        out_type=jax.ShapeDtypeStruct((NC, N_PAD, D), jnp.float32),
        mesh=mesh,
        scratch_types=[
            pltpu.VMEM_SHARED((N_PAD, D), jnp.float32),
            pltpu.VMEM((chunks, B), jnp.int32),
            pltpu.VMEM((B, D), jnp.float32),
        ],
    )


def _dense_body(act, p0r, p1r, c0r, c1r, xr, wlr, blr, wrr, outr):
    cnt = c0r[...][:, 0:1] + c1r[...][:, 0:1]
    inv = 1.0 / jnp.maximum(cnt, 1.0)
    mean = (p0r[...] + p1r[...]) * inv
    y = (jnp.dot(mean, wlr[...], preferred_element_type=jnp.float32)
         + jnp.dot(xr[...], wrr[...], preferred_element_type=jnp.float32)
         + blr[...])
    if act:
        y = jnp.where(y > 0.0, y, jnp.exp(jnp.minimum(y, 0.0)) - 1.0)
    outr[...] = y


def _dense(p0, p1, c0, c1, x, Wl, bl, Wr, act):
    bn = 1000
    grid = (N // bn,)
    row_spec = pl.BlockSpec((bn, D), lambda i: (i, 0))
    cnt_spec = pl.BlockSpec((bn, D), lambda i: (i, 0))
    w_spec = pl.BlockSpec((D, D), lambda i: (0, 0))
    b_spec = pl.BlockSpec((1, D), lambda i: (0, 0))
    return pl.pallas_call(
        functools.partial(_dense_body, act),
        grid=grid,
        in_specs=[row_spec, row_spec, cnt_spec, cnt_spec, row_spec,
                  w_spec, b_spec, w_spec],
        out_specs=row_spec,
        out_shape=jax.ShapeDtypeStruct((N, D), jnp.float32),
    )(p0, p1, c0, c1, x, Wl, bl.reshape(1, D), Wr)


def kernel(x, edge_index, W1l, b1l, W1r, W2l, b2l, W2r):
    src = edge_index[0]
    dst = edge_index[1]
    e = src.shape[0]
    chunks = -(-e // (NW * B))
    pad = chunks * NW * B - e
    if pad:
        src = jnp.concatenate([src, jnp.zeros((pad,), jnp.int32)])
        dst = jnp.concatenate([dst, jnp.full((pad,), N, jnp.int32)])
    src3 = src.reshape(NW, chunks, B)
    dst3 = dst.reshape(NW, chunks, B)
    zf = jnp.zeros((ROWS, D), jnp.float32)
    zc = jnp.zeros((ROWS, D), jnp.float32)
    ones = jnp.ones((B, D), jnp.float32)

    aggregate = _make_aggregate(chunks)
    counts = _make_counts(chunks)

    pcnt = counts(dst3, zc, ones)
    psum = aggregate(x, src3, dst3, zf)
    c0 = pcnt[0, :N]
    c1 = pcnt[1, :N]
    h = _dense(psum[0, :N], psum[1, :N], c0, c1, x, W1l, b1l, W1r, act=True)
    psum2 = aggregate(h, src3, dst3, zf)
    return _dense(psum2[0, :N], psum2[1, :N], c0, c1, h, W2l, b2l, W2r,
                  act=False)
